# baseline (speedup 1.0000x reference)
import numpy as np
import jax
import jax.numpy as jnp
from jax import lax
from jax.experimental import pallas as pl
from jax.experimental.pallas import tpu as pltpu
from jax.experimental.pallas import tpu_sc as plsc

B = 4096
S = 200
D = 64
NW = 32
BB = 128
L = 16
NSLAB = 25
NBLK = 8 * NSLAB


def _positional_encoding_np():
    pos = np.arange(S, dtype=np.float32)[:, None]
    idx = np.arange(D, dtype=np.float32)[None, :]
    angle_rates = 1.0 / np.power(
        np.float32(10000.0), 2.0 * np.floor(idx / 2.0) / np.float32(D)
    )
    angle_rads = (pos * angle_rates).astype(np.float32)
    sines = np.sin(angle_rads[:, 0::2])
    cosines = np.cos(angle_rads[:, 1::2])
    pe = np.concatenate([sines[:, :, None], cosines[:, :, None]], axis=-1)
    return pe.reshape(S, D).astype(np.float32)


_PE = _positional_encoding_np()
_PE_SPLAT = np.repeat(_PE[:, :, None], 16, axis=2)


def _body(table_hbm, x_hbm, pe_hbm, out_hbm,
          ixraw, ix2, p64, g0, g1, o0, o1, pe0, pe1,
          ixs, gs0, gs1, os0, os1, ps0, ps1):
    wid = lax.axis_index("s") * 2 + lax.axis_index("c")
    sid0 = wid * NSLAB

    iota = lax.iota(jnp.int32, L)

    def slab_jb(k):
        sid = sid0 + k
        j0 = pl.multiple_of(lax.shift_left(lax.shift_right_logical(sid, 5), 3), 8)
        b0 = pl.multiple_of(lax.shift_left(lax.bitwise_and(sid, 31), 7), BB)
        return j0, b0

    def block_jb(n):
        j0, b0 = slab_jb(lax.shift_right_logical(n, 3))
        return j0 + lax.bitwise_and(n, 7), b0

    def slab_copy_start(k, q):
        j0, b0 = slab_jb(k)
        pltpu.make_async_copy(
            x_hbm.at[pl.ds(j0, 8), pl.ds(b0, BB)], ixraw.at[q], ixs.at[q]
        ).start()

    def slab_copy_wait(q):
        pltpu.make_async_copy(
            x_hbm.at[pl.ds(0, 8), pl.ds(0, BB)], ixraw.at[q], ixs.at[q]
        ).wait()

    def idx_prep(q, r, p):
        for g in range(BB // L):
            sl = pl.ds(g * L, L)
            v = ixraw[q, r, sl]
            ix2[p, sl] = lax.shift_right_logical(v, 1)
            p64[p, sl] = lax.shift_left(lax.bitwise_and(v, 1), 6)

    def gather_start(gbuf, gsem, p):
        pltpu.make_async_copy(table_hbm.at[ix2.at[p]], gbuf, gsem).start()

    def gather_wait(gbuf, gsem, p):
        pltpu.make_async_copy(table_hbm.at[ix2.at[p]], gbuf, gsem).wait()

    def pe_start(n, pebuf, pesem):
        j, _ = block_jb(n)
        pltpu.make_async_copy(pe_hbm.at[j], pebuf, pesem).start()

    def pe_wait(pebuf, pesem):
        pltpu.make_async_copy(pe_hbm.at[0], pebuf, pesem).wait()

    def out_start(n, obuf, osem):
        j, b0 = block_jb(n)
        pltpu.make_async_copy(
            obuf, out_hbm.at[j, :, pl.ds(b0, BB)], osem
        ).start()

    def out_wait(obuf, osem):
        pltpu.make_async_copy(
            obuf, out_hbm.at[0, :, pl.ds(0, BB)], osem
        ).wait()

    def compute(n, gbuf, obuf, pebuf, p):
        rows = [iota + g * L for g in range(BB // L)]
        cols = [p64[p, pl.ds(g * L, L)] for g in range(BB // L)]

        def c_body(c, carry):
            pe_c = pebuf[c, :]
            for g in range(BB // L):
                val = plsc.load_gather(gbuf, [rows[g], cols[g] + c])
                obuf[c, pl.ds(g * L, L)] = val * 8.0 + pe_c
            return carry

        lax.fori_loop(0, D, c_body, 0)

    gbufs, obufs, gsems, osems = (g0, g1), (o0, o1), (gs0, gs1), (os0, os1)
    pebufs, pesems = (pe0, pe1), (ps0, ps1)

    def block(n, k, r, qcur, qnext):
        p = r % 2
        gbuf, obuf, gsem, osem = gbufs[p], obufs[p], gsems[p], osems[p]
        pebuf, pesem = pebufs[p], pesems[p]

        if r == 0:
            @pl.when(k + 1 < NSLAB)
            def _():
                slab_copy_start(k + 1, qnext)

        gather_wait(gbuf, gsem, p)
        pe_wait(pebuf, pesem)

        @pl.when(n >= 2)
        def _():
            out_wait(obuf, osem)

        compute(n, gbuf, obuf, pebuf, p)
        out_start(n, obuf, osem)

        if r == 6:
            @pl.when(k + 1 < NSLAB)
            def _():
                slab_copy_wait(qnext)

        @pl.when(n + 2 < 8 * NSLAB)
        def _():
            q2, r2 = (qcur, r + 2) if r < 6 else (qnext, r - 6)
            idx_prep(q2, r2, p)
            gather_start(gbuf, gsem, p)
            pe_start(n + 2, pebuf, pesem)

    slab_copy_start(0, 0)
    slab_copy_wait(0)
    for r in range(2):
        idx_prep(0, r, r)
    gather_start(g0, gs0, 0)
    gather_start(g1, gs1, 1)
    pe_start(0, pe0, ps0)
    pe_start(1, pe1, ps1)

    def loop_body(kk, carry):
        for half in range(2):
            k = kk * 2 + half
            for r in range(8):
                block(k * 8 + r, k, r, half, 1 - half)
        return carry

    lax.fori_loop(0, NSLAB // 2, loop_body, 0)
    for r in range(8):
        block(24 * 8 + r, 24, r, 0, 1)
    out_wait(o0, os0)
    out_wait(o1, os1)


_emb_lookup = pl.kernel(
    _body,
    out_type=jax.ShapeDtypeStruct((S, D, B), jnp.float32),
    mesh=plsc.VectorSubcoreMesh(core_axis_name="c", subcore_axis_name="s"),
    scratch_types=[
        pltpu.VMEM((2, 8, BB), jnp.int32),
        pltpu.VMEM((2, BB), jnp.int32),
        pltpu.VMEM((2, BB), jnp.int32),
        pltpu.VMEM((BB, 2 * D), jnp.float32),
        pltpu.VMEM((BB, 2 * D), jnp.float32),
        pltpu.VMEM((D, BB), jnp.float32),
        pltpu.VMEM((D, BB), jnp.float32),
        pltpu.VMEM((D, L), jnp.float32),
        pltpu.VMEM((D, L), jnp.float32),
        pltpu.SemaphoreType.DMA((2,)),
        pltpu.SemaphoreType.DMA,
        pltpu.SemaphoreType.DMA,
        pltpu.SemaphoreType.DMA,
        pltpu.SemaphoreType.DMA,
        pltpu.SemaphoreType.DMA,
        pltpu.SemaphoreType.DMA,
    ],
    compiler_params=pltpu.CompilerParams(
        use_tc_tiling_on_sc=True, needs_layout_passes=False
    ),
)


def kernel(x, table):
    x_t = jnp.transpose(x).astype(jnp.int32)
    table_pairs = table.reshape(500000, 2 * D)
    pe = jnp.asarray(_PE_SPLAT)
    out_t = _emb_lookup(table_pairs, x_t, pe)
    return jnp.transpose(out_t, (2, 0, 1))
